# Initial kernel scaffold; baseline (speedup 1.0000x reference)
import jax
import jax.numpy as jnp
from jax.experimental import pallas as pl


def kernel(x):
    raise NotImplementedError("write your pallas kernel here")


# speedup vs baseline: 1.4198x; 1.4198x over previous
import functools

import jax
import jax.numpy as jnp
from jax import lax
from jax.experimental import pallas as pl
from jax.experimental.pallas import tpu as pltpu

_CORES = 2
_BLK = 16384

_HIGH = lax.Precision.HIGHEST


def _stats_kernel(x_ref, g_ref, s_ref):
    k = pl.program_id(1)

    @pl.when(k == 0)
    def _():
        g_ref[...] = jnp.zeros_like(g_ref)
        s_ref[...] = jnp.zeros_like(s_ref)

    xb = x_ref[...]
    g_ref[...] += lax.dot_general(
        xb, xb, (((0,), (0,)), ((), ())),
        preferred_element_type=jnp.float32)[None]
    s_ref[...] += jnp.sum(xb, axis=0, keepdims=True)[None]


def _factor_kernel(n_rows, g_ref, s_ref, ainv_ref, b_ref, work_ref, nmat_ref):
    c = g_ref.shape[1]
    n = jnp.float32(n_rows)
    g = g_ref[0] + g_ref[1]
    s = s_ref[0] + s_ref[1]
    souter = lax.dot_general(
        s, s, (((0,), (0,)), ((), ())),
        preferred_element_type=jnp.float32, precision=_HIGH)
    work_ref[...] = (g - souter / n) / (n - 1.0)

    lane = lax.broadcasted_iota(jnp.int32, (1, c), 1)

    def chol_step(j, rvec):
        row = work_ref[pl.ds(j, 1), :]
        d = jnp.sum(jnp.where(lane == j, row, 0.0), axis=1,
                    keepdims=True)
        invd = 1.0 / d
        upmask = (lane > j).astype(jnp.float32)
        nmat_ref[pl.ds(j, 1)] = (row * invd * upmask)[None]
        outer = lax.dot_general(row, row, (((0,), (0,)), ((), ())),
                                preferred_element_type=jnp.float32,
                                precision=_HIGH)
        work_ref[...] = work_ref[...] - outer * invd
        return jnp.where(lane == j, lax.rsqrt(d), rvec)

    rvec = lax.fori_loop(0, c, chol_step, jnp.ones((1, c), jnp.float32))

    nu = nmat_ref[...].reshape(c, c)
    rowi = lax.broadcasted_iota(jnp.int32, (c, c), 0)
    coli = lax.broadcasted_iota(jnp.int32, (c, c), 1)
    eye = (rowi == coli).astype(jnp.float32)
    p = eye - nu
    nk = nu
    for _ in range(6):
        nk = lax.dot_general(nk, nk, (((1,), (0,)), ((), ())),
                             preferred_element_type=jnp.float32,
                             precision=_HIGH)
        p = p + lax.dot_general(p, nk, (((1,), (0,)), ((), ())),
                                preferred_element_type=jnp.float32,
                                precision=_HIGH)
    ainv = p * rvec
    ainv_ref[...] = ainv
    b_ref[...] = lax.dot_general(s * (1.0 / n), ainv, (((1,), (0,)), ((), ())),
                                 preferred_element_type=jnp.float32,
                                 precision=_HIGH)


def _apply_kernel(x_ref, a_ref, b_ref, o_ref):
    o_ref[...] = lax.dot_general(
        x_ref[...], a_ref[...], (((1,), (0,)), ((), ())),
        preferred_element_type=jnp.float32) - b_ref[...]


def kernel(x):
    n, c = x.shape
    blk = min(_BLK, n // _CORES)
    kpc = n // (_CORES * blk)

    g_part, s_part = pl.pallas_call(
        _stats_kernel,
        grid=(_CORES, kpc),
        in_specs=[pl.BlockSpec((blk, c), lambda i, k: (i * kpc + k, 0))],
        out_specs=[
            pl.BlockSpec((1, c, c), lambda i, k: (i, 0, 0)),
            pl.BlockSpec((1, 1, c), lambda i, k: (i, 0, 0)),
        ],
        out_shape=[
            jax.ShapeDtypeStruct((_CORES, c, c), jnp.float32),
            jax.ShapeDtypeStruct((_CORES, 1, c), jnp.float32),
        ],
        compiler_params=pltpu.CompilerParams(
            dimension_semantics=("parallel", "arbitrary"),
            vmem_limit_bytes=56 * 1024 * 1024,
        ),
        name="whiten_stats",
    )(x)

    ainv, b = pl.pallas_call(
        functools.partial(_factor_kernel, n),
        out_shape=[
            jax.ShapeDtypeStruct((c, c), jnp.float32),
            jax.ShapeDtypeStruct((1, c), jnp.float32),
        ],
        scratch_shapes=[
            pltpu.VMEM((c, c), jnp.float32),
            pltpu.VMEM((c, 1, c), jnp.float32),
        ],
        name="whiten_factor",
    )(g_part, s_part)

    out = pl.pallas_call(
        _apply_kernel,
        grid=(_CORES, kpc),
        in_specs=[
            pl.BlockSpec((blk, c), lambda i, k: (i * kpc + k, 0)),
            pl.BlockSpec((c, c), lambda i, k: (0, 0)),
            pl.BlockSpec((1, c), lambda i, k: (0, 0)),
        ],
        out_specs=pl.BlockSpec((blk, c), lambda i, k: (i * kpc + k, 0)),
        out_shape=jax.ShapeDtypeStruct((n, c), jnp.float32),
        compiler_params=pltpu.CompilerParams(
            dimension_semantics=("parallel", "arbitrary"),
            vmem_limit_bytes=56 * 1024 * 1024,
        ),
        name="whiten_apply",
    )(x, ainv, b)
    return out


# speedup vs baseline: 1.5357x; 1.0816x over previous
import functools

import jax
import jax.numpy as jnp
from jax import lax
from jax.experimental import pallas as pl
from jax.experimental.pallas import tpu as pltpu

_CORES = 2
_BLK = 16384

_HIGH = lax.Precision.HIGHEST


def _stats_kernel(x_ref, g_ref, s_ref):
    k = pl.program_id(1)

    @pl.when(k == 0)
    def _():
        g_ref[...] = jnp.zeros_like(g_ref)
        s_ref[...] = jnp.zeros_like(s_ref)

    xb = x_ref[...]
    g_ref[...] += lax.dot_general(
        xb, xb, (((0,), (0,)), ((), ())),
        preferred_element_type=jnp.float32)[None]
    s_ref[...] += jnp.sum(xb, axis=0, keepdims=True)[None]


def _factor_kernel(n_rows, g_ref, s_ref, ainv_ref, b_ref):
    c = g_ref.shape[1]
    pb = 8
    n = jnp.float32(n_rows)
    g = g_ref[0] + g_ref[1]
    s = s_ref[0] + s_ref[1]
    souter = lax.dot_general(
        s, s, (((0,), (0,)), ((), ())),
        preferred_element_type=jnp.float32)
    a = (g - souter / n) / (n - 1.0)

    lane = lax.broadcasted_iota(jnp.int32, (1, c), 1)
    dvec = jnp.zeros((1, c), jnp.float32)
    nu_panels = []
    for t in range(c // pb):
        base = t * pb
        rows = [a[base + q:base + q + 1, :] for q in range(pb)]
        scaled = []
        for q in range(pb):
            rq = rows[q]
            d = rq[:, base + q:base + q + 1]
            invd = 1.0 / d
            for r in range(q + 1, pb):
                coeff = rq[:, base + r:base + r + 1]
                rows[r] = rows[r] - rq * (coeff * invd)
            scaled.append(rq * invd)
            dvec = dvec + jnp.where(lane == base + q, rq, 0.0)
        praw = jnp.concatenate(rows, axis=0)
        pscaled = jnp.concatenate(scaled, axis=0)
        lmask = lax.broadcasted_iota(jnp.int32, (pb, c), 1) > (
            lax.broadcasted_iota(jnp.int32, (pb, c), 0) + base)
        nu_panels.append(jnp.where(lmask, pscaled, 0.0))
        a = a - lax.dot_general(pscaled, praw, (((0,), (0,)), ((), ())),
                                preferred_element_type=jnp.float32)

    rvec = lax.rsqrt(dvec)
    nu = jnp.concatenate(nu_panels, axis=0)
    rowi = lax.broadcasted_iota(jnp.int32, (c, c), 0)
    coli = lax.broadcasted_iota(jnp.int32, (c, c), 1)
    eye = (rowi == coli).astype(jnp.float32)
    p = eye - nu
    nk = nu
    for _ in range(6):
        nk = lax.dot_general(nk, nk, (((1,), (0,)), ((), ())),
                             preferred_element_type=jnp.float32)
        p = p + lax.dot_general(p, nk, (((1,), (0,)), ((), ())),
                                preferred_element_type=jnp.float32)
    ainv = p * rvec
    ainv_ref[...] = ainv
    b_ref[...] = lax.dot_general(s * (1.0 / n), ainv, (((1,), (0,)), ((), ())),
                                 preferred_element_type=jnp.float32)


def _apply_kernel(x_ref, a_ref, b_ref, o_ref):
    o_ref[...] = lax.dot_general(
        x_ref[...], a_ref[...], (((1,), (0,)), ((), ())),
        preferred_element_type=jnp.float32) - b_ref[...]


def kernel(x):
    n, c = x.shape
    blk = min(_BLK, n // _CORES)
    kpc = n // (_CORES * blk)
    sblk = min(2 * _BLK, n // _CORES)
    skpc = n // (_CORES * sblk)

    g_part, s_part = pl.pallas_call(
        _stats_kernel,
        grid=(_CORES, skpc),
        in_specs=[pl.BlockSpec((sblk, c), lambda i, k: (i * skpc + k, 0))],
        out_specs=[
            pl.BlockSpec((1, c, c), lambda i, k: (i, 0, 0)),
            pl.BlockSpec((1, 1, c), lambda i, k: (i, 0, 0)),
        ],
        out_shape=[
            jax.ShapeDtypeStruct((_CORES, c, c), jnp.float32),
            jax.ShapeDtypeStruct((_CORES, 1, c), jnp.float32),
        ],
        compiler_params=pltpu.CompilerParams(
            dimension_semantics=("parallel", "arbitrary"),
            vmem_limit_bytes=56 * 1024 * 1024,
        ),
        name="whiten_stats",
    )(x)

    ainv, b = pl.pallas_call(
        functools.partial(_factor_kernel, n),
        out_shape=[
            jax.ShapeDtypeStruct((c, c), jnp.float32),
            jax.ShapeDtypeStruct((1, c), jnp.float32),
        ],
        name="whiten_factor",
    )(g_part, s_part)

    out = pl.pallas_call(
        _apply_kernel,
        grid=(_CORES, kpc),
        in_specs=[
            pl.BlockSpec((blk, c), lambda i, k: (i * kpc + k, 0)),
            pl.BlockSpec((c, c), lambda i, k: (0, 0)),
            pl.BlockSpec((1, c), lambda i, k: (0, 0)),
        ],
        out_specs=pl.BlockSpec((blk, c), lambda i, k: (i * kpc + k, 0)),
        out_shape=jax.ShapeDtypeStruct((n, c), jnp.float32),
        compiler_params=pltpu.CompilerParams(
            dimension_semantics=("parallel", "arbitrary"),
            vmem_limit_bytes=56 * 1024 * 1024,
        ),
        name="whiten_apply",
    )(x, ainv, b)
    return out
